# Initial kernel scaffold; baseline (speedup 1.0000x reference)
import jax
import jax.numpy as jnp
from jax.experimental import pallas as pl


def kernel(feat, edge_index, W0, al0, ar0, W1, al1, ar1):
    raise NotImplementedError("write your pallas kernel here")


# speedup vs baseline: 23.8139x; 23.8139x over previous
import functools

import jax
import jax.numpy as jnp
from jax.experimental import pallas as pl

N = 10000
E = 160000
S = 8


def _mm_kernel(x_ref, w_ref, o_ref):
    o_ref[...] = jnp.dot(x_ref[...], w_ref[...],
                         preferred_element_type=jnp.float32)


def _matmul(x, w, bn):
    n, k = x.shape
    _, m = w.shape
    return pl.pallas_call(
        _mm_kernel,
        grid=(n // bn,),
        in_specs=[pl.BlockSpec((bn, k), lambda i: (i, 0)),
                  pl.BlockSpec((k, m), lambda i: (0, 0))],
        out_specs=pl.BlockSpec((bn, m), lambda i: (i, 0)),
        out_shape=jax.ShapeDtypeStruct((n, m), jnp.float32),
    )(x, w)


def _noise(layer_key, s):
    return 1.0 + jax.random.normal(jax.random.fold_in(layer_key, s),
                                   (E, 1, 1), dtype=jnp.float32)[:, 0, 0]


def kernel(feat, edge_index, W0, al0, ar0, W1, al1, ar1):
    src = edge_index[0]
    dst = edge_index[1]
    nkey = jax.random.key(42)
    k0 = jax.random.fold_in(nkey, 0)
    k1 = jax.random.fold_in(nkey, 1)
    noise0 = jnp.stack([_noise(k0, s) for s in range(S)], axis=1)
    noise1 = jnp.stack([_noise(k1, s) for s in range(S)], axis=1)

    h = _matmul(feat, W0, 1000)
    h3 = h.reshape(N, 8, 8)
    el = jnp.sum(h3 * al0, -1)
    er = jnp.sum(h3 * ar0, -1)
    e = jax.nn.leaky_relu(el[src] + er[dst], 0.2)
    M = jnp.maximum(0.0, el.max(0) + er.max(0))
    ee = jnp.exp(e - M)
    denom = jax.ops.segment_sum(ee, dst, num_segments=N)
    alpha = ee / (denom[dst] + 1e-9)

    msg = (alpha[:, :, None] * h3[src]).reshape(E, 1, 64)
    big = (noise0[:, :, None] * msg).reshape(E, S * 64)
    h0 = jax.ops.segment_sum(big, dst, num_segments=N)
    h0 = jax.nn.elu(h0)

    W1big = jax.scipy.linalg.block_diag(*([W1] * S))
    G = _matmul(h0, W1big, 1000)
    al1b = jnp.tile(al1[:, 0], S)
    ar1b = jnp.tile(ar1[:, 0], S)
    Gl = G * al1b
    Gr = G * ar1b
    e1 = jax.nn.leaky_relu(Gl[src] + Gr[dst], 0.2)
    M1 = jnp.maximum(0.0, Gl.max(0) + Gr.max(0))
    ee1 = jnp.exp(e1 - M1)
    d1 = jax.ops.segment_sum(ee1, dst, num_segments=N)
    alpha1 = ee1 / (d1[dst] + 1e-9)
    noise1b = jnp.repeat(noise1, 7, axis=1)
    out_full = jax.ops.segment_sum(alpha1 * G[src] * noise1b, dst,
                                   num_segments=N)
    return out_full.reshape(N, S, 7).mean(axis=1)


# speedup vs baseline: 29.3024x; 1.2305x over previous
import functools

import jax
import jax.numpy as jnp
from jax import lax
from jax.experimental import pallas as pl
from jax.experimental.pallas import tpu as pltpu
from jax.experimental.pallas import tpu_sc as plsc

N = 10000
E = 160000
S = 8
_NEG = -1e30


def _mm_kernel(x_ref, w_ref, o_ref):
    o_ref[...] = jnp.dot(x_ref[...], w_ref[...],
                         preferred_element_type=jnp.float32)


def _matmul(x, w, bn):
    n, k = x.shape
    _, m = w.shape
    return pl.pallas_call(
        _mm_kernel,
        grid=(n // bn,),
        in_specs=[pl.BlockSpec((bn, k), lambda i: (i, 0)),
                  pl.BlockSpec((k, m), lambda i: (0, 0))],
        out_specs=pl.BlockSpec((bn, m), lambda i: (i, 0)),
        out_shape=jax.ShapeDtypeStruct((n, m), jnp.float32),
    )(x, w)


_B2 = 2000


def _l0_alpha_body(elT, erT, src_h, dst_h, mh, alphaT,
                   colL, colR, den, srcb, dstb, eeb, idxb, mv, sden):
    c = lax.axis_index("c")
    t = lax.axis_index("s")
    hl = t // 4
    h = c * 4 + hl
    q = t % 4
    eq = E // 4
    e0 = q * eq

    def z(i, _):
        den[pl.ds(i * 16, 16)] = jnp.zeros((16,), jnp.float32)
        return 0
    lax.fori_loop(0, 4000 // 16, z, 0)
    @pl.when(t < 10)
    def _():
        pltpu.sync_copy(den.at[pl.ds(0, 4000)], sden.at[pl.ds(t * 4000, 4000)])
    plsc.subcore_barrier()

    pltpu.sync_copy(elT.at[pl.ds(h * N, N)], colL)
    pltpu.sync_copy(erT.at[pl.ds(h * N, N)], colR)
    pltpu.sync_copy(mh, mv)
    M = plsc.load_gather(mv, [jnp.full((16,), h, jnp.int32)])

    hlN = hl * N

    def chunk1(k, _):
        off = e0 + k * _B2
        pltpu.sync_copy(src_h.at[pl.ds(off, _B2)], srcb)
        pltpu.sync_copy(dst_h.at[pl.ds(off, _B2)], dstb)
        def inner(i, _):
            sv = srcb[pl.ds(i * 16, 16)]
            dv = dstb[pl.ds(i * 16, 16)]
            gl = plsc.load_gather(colL, [sv])
            gr = plsc.load_gather(colR, [dv])
            x = gl + gr
            x = jnp.where(x > 0, x, 0.2 * x)
            eeb[pl.ds(i * 16, 16)] = jnp.exp(x - M)
            idxb[pl.ds(i * 16, 16)] = dv + hlN
            return 0
        lax.fori_loop(0, _B2 // 16, inner, 0)
        pltpu.sync_copy(eeb, sden.at[idxb], add=True)
        return 0
    lax.fori_loop(0, eq // _B2, chunk1, 0)
    plsc.subcore_barrier()

    pltpu.sync_copy(sden.at[pl.ds(hlN, N)], den)

    def chunk2(k, _):
        off = e0 + k * _B2
        pltpu.sync_copy(src_h.at[pl.ds(off, _B2)], srcb)
        pltpu.sync_copy(dst_h.at[pl.ds(off, _B2)], dstb)
        def inner(i, _):
            sv = srcb[pl.ds(i * 16, 16)]
            dv = dstb[pl.ds(i * 16, 16)]
            gl = plsc.load_gather(colL, [sv])
            gr = plsc.load_gather(colR, [dv])
            x = gl + gr
            x = jnp.where(x > 0, x, 0.2 * x)
            ee = jnp.exp(x - M)
            dn = plsc.load_gather(den, [dv])
            eeb[pl.ds(i * 16, 16)] = ee / (dn + 1e-9)
            return 0
        lax.fori_loop(0, _B2 // 16, inner, 0)
        pltpu.sync_copy(eeb, alphaT.at[pl.ds(h * E + off, _B2)])
        return 0
    lax.fori_loop(0, eq // _B2, chunk2, 0)


def _l0_alpha(elT, erT, src, dst, M):
    mesh = plsc.VectorSubcoreMesh(core_axis_name="c", subcore_axis_name="s")
    return pl.kernel(
        _l0_alpha_body,
        out_type=jax.ShapeDtypeStruct((8 * E,), jnp.float32),
        mesh=mesh,
        compiler_params=pltpu.CompilerParams(needs_layout_passes=False),
        scratch_types=[
            pltpu.VMEM((N,), jnp.float32),
            pltpu.VMEM((N,), jnp.float32),
            pltpu.VMEM((N,), jnp.float32),
            pltpu.VMEM((_B2,), jnp.int32),
            pltpu.VMEM((_B2,), jnp.int32),
            pltpu.VMEM((_B2,), jnp.float32),
            pltpu.VMEM((_B2,), jnp.int32),
            pltpu.VMEM((16,), jnp.float32),
            pltpu.VMEM_SHARED((4 * N,), jnp.float32),
        ],
    )(elT, erT, src, dst, M)


def _noise(layer_key, s):
    return 1.0 + jax.random.normal(jax.random.fold_in(layer_key, s),
                                   (E, 1, 1), dtype=jnp.float32)[:, 0, 0]


def kernel(feat, edge_index, W0, al0, ar0, W1, al1, ar1):
    src = edge_index[0]
    dst = edge_index[1]
    nkey = jax.random.key(42)
    k0 = jax.random.fold_in(nkey, 0)
    k1 = jax.random.fold_in(nkey, 1)
    noise0 = jnp.stack([_noise(k0, s) for s in range(S)], axis=1)
    noise1 = jnp.stack([_noise(k1, s) for s in range(S)], axis=1)

    h = _matmul(feat, W0, 1000)
    h3 = h.reshape(N, 8, 8)
    el = jnp.sum(h3 * al0, -1)
    er = jnp.sum(h3 * ar0, -1)
    M0 = jnp.zeros(16).at[:8].set(jnp.maximum(0.0, el.max(0) + er.max(0)))
    alphaT = _l0_alpha(el.T.reshape(-1), er.T.reshape(-1), src, dst, M0)
    alpha = alphaT.reshape(8, E).T

    msg = (alpha[:, :, None] * h3[src]).reshape(E, 1, 64)
    big = (noise0[:, :, None] * msg).reshape(E, S * 64)
    h0 = jax.ops.segment_sum(big, dst, num_segments=N)
    h0 = jax.nn.elu(h0)

    W1big = jax.scipy.linalg.block_diag(*([W1] * S))
    G = _matmul(h0, W1big, 1000)
    al1b = jnp.tile(al1[:, 0], S)
    ar1b = jnp.tile(ar1[:, 0], S)
    Gl = G * al1b
    Gr = G * ar1b
    e1 = jax.nn.leaky_relu(Gl[src] + Gr[dst], 0.2)
    M1 = jnp.maximum(0.0, Gl.max(0) + Gr.max(0))
    ee1 = jnp.exp(e1 - M1)
    d1 = jax.ops.segment_sum(ee1, dst, num_segments=N)
    alpha1 = ee1 / (d1[dst] + 1e-9)
    noise1b = jnp.repeat(noise1, 7, axis=1)
    out_full = jax.ops.segment_sum(alpha1 * G[src] * noise1b, dst,
                                   num_segments=N)
    return out_full.reshape(N, S, 7).mean(axis=1)
